# baseline (speedup 1.0000x reference)
import jax
import jax.numpy as jnp
from jax import lax
from jax.experimental import pallas as pl
from jax.experimental.pallas import tpu as pltpu

_B, _N, _Q = 16, 2048, 128
_CHUNK = 128
_K = _N // _CHUNK
_GB = 2
_STEPS = _B // _GB


def _fused_body(xy_ref, s_ref, w_ref, b_ref, out_ref, p_ref):
    g = pl.program_id(0)
    sc = s_ref[...]
    v = xy_ref[...]
    for j in range(_GB):
        acc = None
        for k in range(_K):
            xc = lax.slice(v, (32 * j + k, 0), (32 * j + k + 1, _CHUNK))
            yc = lax.slice(v, (32 * j + 16 + k, 0), (32 * j + 17 + k, _CHUNK))
            phi = jnp.maximum(jnp.minimum(sc - xc, yc - sc), 0.0)
            acc = phi if acc is None else acc + phi
        pooled = jnp.sum(acc, axis=1, keepdims=True)
        p_ref[pl.ds(g * _GB + j, 1), :] = pooled.reshape(1, _Q)

    @pl.when(g == _STEPS - 1)
    def _():
        r = lax.dot_general(
            p_ref[...], w_ref[...], (((1,), (1,)), ((), ())),
            preferred_element_type=jnp.float32,
        )
        out_ref[...] = jnp.maximum(r + b_ref[...], 0.0)


_fused = pl.pallas_call(
    _fused_body,
    grid=(_STEPS,),
    in_specs=[
        pl.BlockSpec((32 * _GB, _CHUNK), lambda g: (g, 0)),
        pl.BlockSpec((_Q, 1), lambda g: (0, 0)),
        pl.BlockSpec((_Q, _Q), lambda g: (0, 0)),
        pl.BlockSpec((1, _Q), lambda g: (0, 0)),
    ],
    out_specs=pl.BlockSpec((_B, _Q), lambda g: (0, 0)),
    out_shape=jax.ShapeDtypeStruct((_B, _Q), jnp.float32),
    scratch_shapes=[pltpu.VMEM((_B, _Q), jnp.float32)],
)


def kernel(diagram, samples, rho_w, rho_b):
    xy = diagram.transpose(0, 2, 1).reshape(32 * _B, _CHUNK)
    return _fused(xy, samples.reshape(_Q, 1), rho_w, rho_b.reshape(1, _Q))


# speedup vs baseline: 1.2375x; 1.2375x over previous
import jax
import jax.numpy as jnp
from jax import lax
from jax.experimental import pallas as pl
from jax.experimental.pallas import tpu as pltpu

_B, _N, _Q = 16, 2048, 128
_CHUNK = 128
_K = _N // _CHUNK
_GB = 8
_STEPS = _B // _GB


def _fused_body(xy_ref, s_ref, w_ref, b_ref, out_ref, p_ref):
    g = pl.program_id(0)
    sc = s_ref[...]
    v = xy_ref[...]
    for j in range(_GB):
        acc = None
        for k in range(_K):
            xc = lax.slice(v, (32 * j + k, 0), (32 * j + k + 1, _CHUNK))
            yc = lax.slice(v, (32 * j + 16 + k, 0), (32 * j + 17 + k, _CHUNK))
            phi = jnp.maximum(jnp.minimum(sc - xc, yc - sc), 0.0)
            acc = phi if acc is None else acc + phi
        pooled = jnp.sum(acc, axis=1, keepdims=True)
        p_ref[pl.ds(g * _GB + j, 1), :] = pooled.reshape(1, _Q)

    @pl.when(g == _STEPS - 1)
    def _():
        r = lax.dot_general(
            p_ref[...], w_ref[...], (((1,), (1,)), ((), ())),
            preferred_element_type=jnp.float32,
        )
        out_ref[...] = jnp.maximum(r + b_ref[...], 0.0)


_fused = pl.pallas_call(
    _fused_body,
    grid=(_STEPS,),
    in_specs=[
        pl.BlockSpec((32 * _GB, _CHUNK), lambda g: (g, 0)),
        pl.BlockSpec((_Q, 1), lambda g: (0, 0)),
        pl.BlockSpec((_Q, _Q), lambda g: (0, 0)),
        pl.BlockSpec((1, _Q), lambda g: (0, 0)),
    ],
    out_specs=pl.BlockSpec((_B, _Q), lambda g: (0, 0)),
    out_shape=jax.ShapeDtypeStruct((_B, _Q), jnp.float32),
    scratch_shapes=[pltpu.VMEM((_B, _Q), jnp.float32)],
)


def kernel(diagram, samples, rho_w, rho_b):
    xy = diagram.transpose(0, 2, 1).reshape(32 * _B, _CHUNK)
    return _fused(xy, samples.reshape(_Q, 1), rho_w, rho_b.reshape(1, _Q))


# speedup vs baseline: 1.2576x; 1.0162x over previous
import jax
import jax.numpy as jnp
from jax import lax
from jax.experimental import pallas as pl
from jax.experimental.pallas import tpu as pltpu

_B, _N, _Q = 16, 2048, 128
_CHUNK = 128
_K = _N // _CHUNK
_GB = 16
_STEPS = _B // _GB


def _fused_body(xy_ref, s_ref, w_ref, b_ref, out_ref, p_ref):
    g = pl.program_id(0)
    sc = s_ref[...]
    v = xy_ref[...]
    for j in range(_GB):
        acc = None
        for k in range(_K):
            xc = lax.slice(v, (32 * j + k, 0), (32 * j + k + 1, _CHUNK))
            yc = lax.slice(v, (32 * j + 16 + k, 0), (32 * j + 17 + k, _CHUNK))
            phi = jnp.maximum(jnp.minimum(sc - xc, yc - sc), 0.0)
            acc = phi if acc is None else acc + phi
        pooled = jnp.sum(acc, axis=1, keepdims=True)
        p_ref[pl.ds(g * _GB + j, 1), :] = pooled.reshape(1, _Q)

    @pl.when(g == _STEPS - 1)
    def _():
        r = lax.dot_general(
            p_ref[...], w_ref[...], (((1,), (1,)), ((), ())),
            preferred_element_type=jnp.float32,
        )
        out_ref[...] = jnp.maximum(r + b_ref[...], 0.0)


_fused = pl.pallas_call(
    _fused_body,
    grid=(_STEPS,),
    in_specs=[
        pl.BlockSpec((32 * _GB, _CHUNK), lambda g: (g, 0)),
        pl.BlockSpec((_Q, 1), lambda g: (0, 0)),
        pl.BlockSpec((_Q, _Q), lambda g: (0, 0)),
        pl.BlockSpec((1, _Q), lambda g: (0, 0)),
    ],
    out_specs=pl.BlockSpec((_B, _Q), lambda g: (0, 0)),
    out_shape=jax.ShapeDtypeStruct((_B, _Q), jnp.float32),
    scratch_shapes=[pltpu.VMEM((_B, _Q), jnp.float32)],
)


def kernel(diagram, samples, rho_w, rho_b):
    xy = diagram.transpose(0, 2, 1).reshape(32 * _B, _CHUNK)
    return _fused(xy, samples.reshape(_Q, 1), rho_w, rho_b.reshape(1, _Q))
